# baseline (speedup 1.0000x reference)
import functools

import jax
import jax.numpy as jnp
from jax import lax
from jax.experimental import pallas as pl
from jax.experimental.pallas import tpu as pltpu
from jax.experimental.pallas import tpu_sc as plsc

ROWS = 64
COLS = 32768
LANES = 16
CHUNKS = COLS // LANES
UNROLL = 4
SEG_BOUNDS_PER_ROW = (
    (0, 16384, 32768),
    (0, 16384, 30720, 32768),
)
NSEMS = sum(len(b) - 1 for b in SEG_BOUNDS_PER_ROW)
ROWS_PER_W = 2
WORKERS = ROWS // ROWS_PER_W

_mesh = plsc.VectorSubcoreMesh(core_axis_name="c", subcore_axis_name="s")


def _shuffle(x, perm):
    return x.at[perm].get(mode="promise_in_bounds")


def _row_argmin(row_ref, lane, copies, seg_bounds):
    minvs = tuple(jnp.full((LANES,), jnp.inf, jnp.float32) for _ in range(UNROLL))
    minis = tuple(jnp.zeros((LANES,), jnp.int32) for _ in range(UNROLL))

    for seg in range(len(seg_bounds) - 1):
        copies[seg].wait()
        iter_base = seg_bounds[seg] // (UNROLL * LANES)
        iter_end = seg_bounds[seg + 1] // (UNROLL * LANES)

        def body(i, carry, _iter_base=iter_base):
            mvs, mis = carry
            i_abs = i + _iter_base
            base = i_abs * (UNROLL * LANES)
            ivec = jnp.full((LANES,), 0, jnp.int32) + i_abs
            nv, ni = [], []
            for u in range(UNROLL):
                v = row_ref[pl.ds(base + u * LANES, LANES)]
                lt = v < mvs[u]
                nv.append(jnp.where(lt, v, mvs[u]))
                ni.append(jnp.where(lt, ivec, mis[u]))
            return tuple(nv), tuple(ni)

        minvs, minis = lax.fori_loop(0, iter_end - iter_base, body, (minvs, minis))

    mv = minvs[0]
    mi = minis[0] * (UNROLL * LANES) + lane
    for u in range(1, UNROLL):
        idx_u = minis[u] * (UNROLL * LANES) + (lane + u * LANES)
        better = (minvs[u] < mv) | ((minvs[u] == mv) & (idx_u < mi))
        mv = jnp.where(better, minvs[u], mv)
        mi = jnp.where(better, idx_u, mi)

    for off in (8, 4, 2, 1):
        perm = lane ^ off
        mv2 = _shuffle(mv, perm)
        mi2 = _shuffle(mi, perm)
        better = (mv2 < mv) | ((mv2 == mv) & (mi2 < mi))
        mv = jnp.where(better, mv2, mv)
        mi = jnp.where(better, mi2, mi)
    return mi


@functools.partial(
    pl.kernel,
    out_type=jax.ShapeDtypeStruct((WORKERS, LANES), jnp.float32),
    mesh=_mesh,
    scratch_types=[
        pltpu.VMEM((ROWS_PER_W, COLS), jnp.float32),
        pltpu.VMEM((LANES,), jnp.float32),
    ]
    + [pltpu.SemaphoreType.DMA] * NSEMS,
)
def _argmin_sc(in_hbm, out_hbm, rows_v, out_v, *sems):
    c = lax.axis_index("c")
    s = lax.axis_index("s")
    wid = s * 2 + c
    r0 = wid * ROWS_PER_W
    lane = lax.iota(jnp.int32, LANES)

    copies = []
    sem_i = 0
    for j in range(ROWS_PER_W):
        bounds = SEG_BOUNDS_PER_ROW[j]
        row_copies = []
        for seg in range(len(bounds) - 1):
            lo, hi = bounds[seg], bounds[seg + 1]
            cp = pltpu.async_copy(
                in_hbm.at[r0 + j, pl.ds(lo, hi - lo)],
                rows_v.at[j, pl.ds(lo, hi - lo)],
                sems[sem_i],
            )
            sem_i += 1
            row_copies.append(cp)
        copies.append(row_copies)

    b0 = _row_argmin(rows_v.at[0], lane, copies[0], SEG_BOUNDS_PER_ROW[0])
    b1 = _row_argmin(rows_v.at[1], lane, copies[1], SEG_BOUNDS_PER_ROW[1])

    outvec = jnp.where(
        lane == 0,
        b0.astype(jnp.float32),
        jnp.where(lane == 1, b1.astype(jnp.float32), jnp.float32(0.0)),
    )
    out_v[...] = outvec
    pltpu.sync_copy(out_v, out_hbm.at[wid])


def kernel(inputs):
    padded = _argmin_sc(inputs)
    return padded[:, :ROWS_PER_W].reshape(ROWS, 1)


# speedup vs baseline: 1.1313x; 1.1313x over previous
import functools

import jax
import jax.numpy as jnp
from jax import lax
from jax.experimental import pallas as pl
from jax.experimental.pallas import tpu as pltpu
from jax.experimental.pallas import tpu_sc as plsc

ROWS = 64
COLS = 32768
LANES = 16
UNROLL = 8
SC_ROWS = 32
TC_ROWS = ROWS - SC_ROWS
TC_BLOCK_ROWS = 8
SEG_BOUNDS = (0, 16384, 30720, 32768)
NSEGS = len(SEG_BOUNDS) - 1

_mesh = plsc.VectorSubcoreMesh(core_axis_name="c", subcore_axis_name="s")


def _shuffle(x, perm):
    return x.at[perm].get(mode="promise_in_bounds")


def _row_argmin(row_ref, lane, copies):
    minvs = tuple(jnp.full((LANES,), jnp.inf, jnp.float32) for _ in range(UNROLL))
    minis = tuple(jnp.zeros((LANES,), jnp.int32) for _ in range(UNROLL))

    for seg in range(NSEGS):
        copies[seg].wait()
        iter_base = SEG_BOUNDS[seg] // (UNROLL * LANES)
        iter_end = SEG_BOUNDS[seg + 1] // (UNROLL * LANES)

        def body(i, carry, _iter_base=iter_base):
            mvs, mis = carry
            i_abs = i + _iter_base
            base = i_abs * (UNROLL * LANES)
            ivec = jnp.full((LANES,), 0, jnp.int32) + i_abs
            nv, ni = [], []
            for u in range(UNROLL):
                v = row_ref[pl.ds(base + u * LANES, LANES)]
                lt = v < mvs[u]
                nv.append(jnp.where(lt, v, mvs[u]))
                ni.append(jnp.where(lt, ivec, mis[u]))
            return tuple(nv), tuple(ni)

        minvs, minis = lax.fori_loop(0, iter_end - iter_base, body, (minvs, minis))

    mv = minvs[0]
    mi = minis[0] * (UNROLL * LANES) + lane
    for u in range(1, UNROLL):
        idx_u = minis[u] * (UNROLL * LANES) + (lane + u * LANES)
        better = (minvs[u] < mv) | ((minvs[u] == mv) & (idx_u < mi))
        mv = jnp.where(better, minvs[u], mv)
        mi = jnp.where(better, idx_u, mi)

    for off in (8, 4, 2, 1):
        perm = lane ^ off
        mv2 = _shuffle(mv, perm)
        mi2 = _shuffle(mi, perm)
        better = (mv2 < mv) | ((mv2 == mv) & (mi2 < mi))
        mv = jnp.where(better, mv2, mv)
        mi = jnp.where(better, mi2, mi)
    return mi


@functools.partial(
    pl.kernel,
    out_type=jax.ShapeDtypeStruct((SC_ROWS, LANES), jnp.float32),
    mesh=_mesh,
    scratch_types=[
        pltpu.VMEM((COLS,), jnp.float32),
        pltpu.VMEM((LANES,), jnp.float32),
    ]
    + [pltpu.SemaphoreType.DMA] * NSEGS,
)
def _argmin_sc(in_hbm, out_hbm, row_v, out_v, *sems):
    c = lax.axis_index("c")
    s = lax.axis_index("s")
    wid = s * 2 + c
    lane = lax.iota(jnp.int32, LANES)

    copies = []
    for seg in range(NSEGS):
        lo, hi = SEG_BOUNDS[seg], SEG_BOUNDS[seg + 1]
        copies.append(
            pltpu.async_copy(
                in_hbm.at[wid, pl.ds(lo, hi - lo)],
                row_v.at[pl.ds(lo, hi - lo)],
                sems[seg],
            )
        )

    b = _row_argmin(row_v, lane, copies)
    outvec = jnp.where(lane == 0, b.astype(jnp.float32), jnp.float32(0.0))
    out_v[...] = outvec
    pltpu.sync_copy(out_v, out_hbm.at[wid])


def _tc_body(x_ref, o_ref):
    v = x_ref[...]
    m = jnp.min(v, axis=1, keepdims=True)
    iota = lax.broadcasted_iota(jnp.int32, v.shape, 1)
    idx = jnp.where(v == m, iota, jnp.int32(COLS))
    a = jnp.min(idx, axis=1, keepdims=True)
    o_ref[...] = a.astype(jnp.float32)


_argmin_tc = pl.pallas_call(
    _tc_body,
    out_shape=jax.ShapeDtypeStruct((TC_ROWS, 1), jnp.float32),
    grid=(TC_ROWS // TC_BLOCK_ROWS,),
    in_specs=[
        pl.BlockSpec(
            (TC_BLOCK_ROWS, COLS),
            lambda i: (i + SC_ROWS // TC_BLOCK_ROWS, 0),
        )
    ],
    out_specs=pl.BlockSpec((TC_BLOCK_ROWS, 1), lambda i: (i, 0)),
)


def kernel(inputs):
    top = _argmin_sc(inputs)[:, :1]
    bot = _argmin_tc(inputs)
    return jnp.concatenate([top, bot], axis=0)
